# baseline (speedup 1.0000x reference)
import jax
import jax.numpy as jnp
from jax import lax
from jax.experimental import pallas as pl
from jax.experimental.pallas import tpu as pltpu
from jax.experimental.pallas import tpu_sc as plsc

_HEADS = 8
_OUT_C = 8
_HC = _HEADS * _OUT_C
_ROW = _HC + 2 * _OUT_C

_NC = 2
_NS = 16
_NW = _NC * _NS
_B = 128
_NPAD = 10240
_BN = 2000


def _prologue_body(x_ref, w_ref, asrc_ref, adst_ref,
                   xs_ref, dtab_ref, gv_ref, acc_ref):
    i = pl.program_id(0)
    xw = jnp.dot(x_ref[...], w_ref[...],
                 preferred_element_type=jnp.float32)
    n = xw.shape[0]
    xw3 = xw.reshape(n, _HEADS, _OUT_C)
    a_src = jnp.sum(xw3 * asrc_ref[...][None], axis=-1)
    a_dst = jnp.sum(xw3 * adst_ref[...][None], axis=-1)
    xs_ref[...] = jnp.concatenate(
        [xw, a_src, jnp.zeros_like(a_src)], axis=1)
    dtab_ref[...] = jnp.concatenate([a_dst, a_dst], axis=1)
    m = jnp.max(a_src, axis=0, keepdims=True)

    @pl.when(i == 0)
    def _():
        acc_ref[...] = m

    @pl.when(i > 0)
    def _():
        acc_ref[...] = jnp.maximum(acc_ref[...], m)

    @pl.when(i == pl.num_programs(0) - 1)
    def _():
        gv_ref[...] = jnp.concatenate(
            [jnp.zeros((1, _HEADS), jnp.float32), acc_ref[...]], axis=1)


def _prologue(x, w, att_src, att_dst):
    n = x.shape[0]
    nblk = n // _BN
    return pl.pallas_call(
        _prologue_body,
        grid=(nblk,),
        in_specs=[
            pl.BlockSpec((_BN, x.shape[1]), lambda i: (i, 0)),
            pl.BlockSpec(w.shape, lambda i: (0, 0)),
            pl.BlockSpec(att_src.shape, lambda i: (0, 0)),
            pl.BlockSpec(att_dst.shape, lambda i: (0, 0)),
        ],
        out_specs=(
            pl.BlockSpec((_BN, _ROW), lambda i: (i, 0)),
            pl.BlockSpec((_BN, 2 * _HEADS), lambda i: (i, 0)),
            pl.BlockSpec((1, 2 * _HEADS), lambda i: (0, 0)),
        ),
        out_shape=(
            jax.ShapeDtypeStruct((n, _ROW), jnp.float32),
            jax.ShapeDtypeStruct((n, 2 * _HEADS), jnp.float32),
            jax.ShapeDtypeStruct((1, 2 * _HEADS), jnp.float32),
        ),
        scratch_shapes=[pltpu.VMEM((1, _HEADS), jnp.float32)],
    )(x, w, att_src, att_dst)


def _edge_body(xs_hbm, dtab_hbm, gv_hbm, src_hbm, dst_hbm, acc_hbm,
               sidx0, sidx1, didx0, didx1, sdix0, sdix1, xr0, xr1, dt0, dt1,
               mw0, mw1, gv_v, za_v, acc_s,
               isem0, isem1, gsem0, gsem1, ssem0, ssem1):
    c = lax.axis_index("c")
    s = lax.axis_index("s")
    wid = s * _NC + c
    rows_per_tile = _NPAD // _NS
    ntot = src_hbm.shape[0] // _B
    nbase = ntot // _NW
    nxtra = ntot - nbase * _NW
    nch = jnp.where(wid < nxtra, nbase + 1, nbase)

    sidx = (sidx0, sidx1)
    didx = (didx0, didx1)
    sdix = (sdix0, sdix1)
    xr = (xr0, xr1)
    dt = (dt0, dt1)
    mw = (mw0, mw1)
    isem = (isem0, isem1)
    gsem = (gsem0, gsem1)
    ssem = (ssem0, ssem1)

    zrows = 64
    def _zrow(i, _):
        for j in range(5):
            za_v[i, pl.ds(16 * j, 16)] = jnp.zeros((16,), jnp.float32)
        return 0
    lax.fori_loop(0, zrows, _zrow, 0)
    row0 = s * rows_per_tile

    def _zcopy(i, _):
        pltpu.sync_copy(za_v, acc_s.at[pl.ds(row0 + i * zrows, zrows)])
        return 0
    lax.fori_loop(0, rows_per_tile // zrows, _zcopy, 0)
    pltpu.sync_copy(gv_hbm, gv_v)
    plsc.subcore_barrier()

    zero16 = jnp.zeros((16,), jnp.int32)
    iota16 = lax.iota(jnp.int32, 16)
    lane8 = iota16 // 8
    lo8 = iota16 & 7
    g2 = plsc.load_gather(gv_v, [zero16, lo8 + 8])

    def _issue_idx(i, p):
        ci = jnp.where(i < nbase, wid * nbase + i, nbase * _NW + wid)
        base = ci * _B
        pltpu.async_copy(src_hbm.at[pl.ds(base, _B)], sidx[p], isem[p])
        pltpu.async_copy(dst_hbm.at[pl.ds(base, _B)], didx[p], isem[p])

    def _wait_idx(p):
        pltpu.make_async_copy(src_hbm.at[pl.ds(0, _B)], sidx[p], isem[p]).wait()
        pltpu.make_async_copy(dst_hbm.at[pl.ds(0, _B)], didx[p], isem[p]).wait()

    def _issue_gather(p):
        pltpu.async_copy(xs_hbm.at[sidx[p]], xr[p], gsem[p])
        pltpu.async_copy(dtab_hbm.at[didx[p]], dt[p], gsem[p])

    def _wait_gather(p):
        pltpu.make_async_copy(xs_hbm.at[sidx[p]], xr[p], gsem[p]).wait()
        pltpu.make_async_copy(dtab_hbm.at[didx[p]], dt[p], gsem[p]).wait()

    def _issue_scatter(p):
        pltpu.async_copy(mw[p], acc_s.at[sdix[p]], ssem[p], add=True)

    def _wait_scatter(p):
        pltpu.make_async_copy(mw[p], acc_s.at[sdix[p]], ssem[p]).wait()

    def _snap_didx(p):
        for k in range(_B // 16):
            sdix[p][pl.ds(16 * k, 16)] = didx[p][pl.ds(16 * k, 16)]

    def _compute(p):
        xr_v, dt_v, mw_v = xr[p], dt[p], mw[p]

        def _edge(e4, _):
            for u in range(2):
                e = e4 * 4 + 2 * u
                epair = e + lane8
                sv = plsc.load_gather(xr_v, [epair, lo8 + _HC])
                dv = plsc.load_gather(dt_v, [epair, lo8])
                t = sv + dv
                t = jnp.maximum(t, 0.2 * t)
                q = dv + g2
                ub = jnp.maximum(q, 0.2 * q)
                w2 = jnp.exp(t - ub)
                plsc.store_scatter(mw_v, [epair, lo8 + _HC], w2)
            for v in range(4):
                e = e4 * 4 + v
                erow = jnp.full((16,), e, jnp.int32)
                for k in range(4):
                    wk = plsc.load_gather(mw_v, [erow, _HC + lane8 + 2 * k])
                    mw_v[e, pl.ds(16 * k, 16)] = xr_v[e, pl.ds(16 * k, 16)] * wk
            return 0
        lax.fori_loop(0, _B // 4, _edge, 0)

    _issue_idx(0, 0)
    _issue_idx(1, 1)
    _wait_idx(0)
    _issue_gather(0)

    def _step(i, p):
        q = 1 - p
        _wait_gather(p)

        @pl.when(i + 1 < nch)
        def _():
            _wait_idx(q)
            _issue_gather(q)

        @pl.when(i >= 2)
        def _():
            _wait_scatter(p)
        _snap_didx(p)

        @pl.when(i + 2 < nch)
        def _():
            _issue_idx(i + 2, p)
        _compute(p)
        _issue_scatter(p)

    def _pair(j, _):
        _step(2 * j, 0)
        _step(2 * j + 1, 1)
        return 0
    lax.fori_loop(0, nbase // 2, _pair, 0)

    @pl.when(wid < nxtra)
    def _():
        _step(nbase, 0)
    _wait_scatter(0)
    _wait_scatter(1)
    plsc.subcore_barrier()

    pltpu.sync_copy(acc_s.at[pl.ds(row0, rows_per_tile)],
                    acc_hbm.at[c, pl.ds(row0, rows_per_tile)])


def _edge_phase(xs, dtab, gv, src, dst):
    rows_per_tile = _NPAD // _NS
    mesh = plsc.VectorSubcoreMesh(core_axis_name="c", subcore_axis_name="s")
    kern = pl.kernel(
        _edge_body,
        out_type=jax.ShapeDtypeStruct((_NC, _NPAD, _ROW), jnp.float32),
        mesh=mesh,
        compiler_params=pltpu.CompilerParams(use_tc_tiling_on_sc=False, needs_layout_passes=False),
        scratch_types=(
            [pltpu.VMEM((_B,), jnp.int32)] * 6
            + [pltpu.VMEM((_B, _ROW), jnp.float32)] * 2
            + [pltpu.VMEM((_B, 16), jnp.float32)] * 2
            + [pltpu.VMEM((_B, _ROW), jnp.float32)] * 2
            + [
                pltpu.VMEM((1, 16), jnp.float32),
                pltpu.VMEM((64, _ROW), jnp.float32),
                pltpu.VMEM_SHARED((_NPAD, _ROW), jnp.float32),
            ]
            + [pltpu.SemaphoreType.DMA] * 6
        ),
    )
    return kern(xs, dtab, gv, src, dst)


def _epilogue_body(acc_ref, bias_ref, out_ref):
    acc = acc_ref[0] + acc_ref[1]
    n = acc.shape[0]
    agg = acc[:, :_HC]
    den = acc[:, _HC:_HC + _HEADS]
    d8 = den.reshape(n, _HEADS, 1) + 1e-16
    a3 = agg.reshape(n, _HEADS, _OUT_C)
    out = (a3 / d8).reshape(n, _HC) + bias_ref[...][None]
    out_ref[...] = jnp.maximum(out, 0.0)


def _epilogue(acc2, bias):
    bn = 1280
    nblk = _NPAD // bn
    return pl.pallas_call(
        _epilogue_body,
        grid=(nblk,),
        in_specs=[
            pl.BlockSpec((_NC, bn, _ROW), lambda i: (0, i, 0)),
            pl.BlockSpec(bias.shape, lambda i: (0,)),
        ],
        out_specs=pl.BlockSpec((bn, _HC), lambda i: (i, 0)),
        out_shape=jax.ShapeDtypeStruct((_NPAD, _HC), jnp.float32),
    )(acc2, bias)


@jax.jit
def kernel(x, edge_index, W, att_src, att_dst, bias):
    src = edge_index[0].astype(jnp.int32)
    dst = edge_index[1].astype(jnp.int32)
    xs, dtab, gv = _prologue(x, W, att_src, att_dst)
    acc2 = _edge_phase(xs, dtab, gv, src, dst)
    out = _epilogue(acc2, bias)
    return out[:x.shape[0]]


# speedup vs baseline: 1.0066x; 1.0066x over previous
import jax
import jax.numpy as jnp
from jax import lax
from jax.experimental import pallas as pl
from jax.experimental.pallas import tpu as pltpu
from jax.experimental.pallas import tpu_sc as plsc

_HEADS = 8
_OUT_C = 8
_HC = _HEADS * _OUT_C
_ROW = _HC + 2 * _OUT_C

_NC = 2
_NS = 16
_NW = _NC * _NS
_B = 80
_NPAD = 10240
_BN = 2000


def _prologue_body(x_ref, w_ref, asrc_ref, adst_ref,
                   xs_ref, dtab_ref, gv_ref, acc_ref):
    i = pl.program_id(0)
    xw = jnp.dot(x_ref[...], w_ref[...],
                 preferred_element_type=jnp.float32)
    n = xw.shape[0]
    xw3 = xw.reshape(n, _HEADS, _OUT_C)
    a_src = jnp.sum(xw3 * asrc_ref[...][None], axis=-1)
    a_dst = jnp.sum(xw3 * adst_ref[...][None], axis=-1)
    xs_ref[...] = jnp.concatenate(
        [xw, a_src, jnp.zeros_like(a_src)], axis=1)
    dtab_ref[...] = jnp.concatenate([a_dst, a_dst], axis=1)
    m = jnp.max(a_src, axis=0, keepdims=True)

    @pl.when(i == 0)
    def _():
        acc_ref[...] = m

    @pl.when(i > 0)
    def _():
        acc_ref[...] = jnp.maximum(acc_ref[...], m)

    @pl.when(i == pl.num_programs(0) - 1)
    def _():
        gv_ref[...] = jnp.concatenate(
            [jnp.zeros((1, _HEADS), jnp.float32), acc_ref[...]], axis=1)


def _prologue(x, w, att_src, att_dst):
    n = x.shape[0]
    nblk = n // _BN
    return pl.pallas_call(
        _prologue_body,
        grid=(nblk,),
        in_specs=[
            pl.BlockSpec((_BN, x.shape[1]), lambda i: (i, 0)),
            pl.BlockSpec(w.shape, lambda i: (0, 0)),
            pl.BlockSpec(att_src.shape, lambda i: (0, 0)),
            pl.BlockSpec(att_dst.shape, lambda i: (0, 0)),
        ],
        out_specs=(
            pl.BlockSpec((_BN, _ROW), lambda i: (i, 0)),
            pl.BlockSpec((_BN, 2 * _HEADS), lambda i: (i, 0)),
            pl.BlockSpec((1, 2 * _HEADS), lambda i: (0, 0)),
        ),
        out_shape=(
            jax.ShapeDtypeStruct((n, _ROW), jnp.float32),
            jax.ShapeDtypeStruct((n, 2 * _HEADS), jnp.float32),
            jax.ShapeDtypeStruct((1, 2 * _HEADS), jnp.float32),
        ),
        scratch_shapes=[pltpu.VMEM((1, _HEADS), jnp.float32)],
    )(x, w, att_src, att_dst)


def _edge_body(xs_hbm, dtab_hbm, gv_hbm, src_hbm, dst_hbm, acc_hbm,
               sidx0, sidx1, didx0, didx1, sdix0, sdix1, xr0, xr1, dt0, dt1,
               mw0, mw1, gv_v, za_v, acc_s,
               isem0, isem1, gsem0, gsem1, ssem0, ssem1):
    c = lax.axis_index("c")
    s = lax.axis_index("s")
    wid = s * _NC + c
    rows_per_tile = _NPAD // _NS
    ep = src_hbm.shape[0] // _NW
    nch = ep // _B
    e0 = wid * ep

    sidx = (sidx0, sidx1)
    didx = (didx0, didx1)
    sdix = (sdix0, sdix1)
    xr = (xr0, xr1)
    dt = (dt0, dt1)
    mw = (mw0, mw1)
    isem = (isem0, isem1)
    gsem = (gsem0, gsem1)
    ssem = (ssem0, ssem1)

    zrows = 64
    def _zrow(i, _):
        for j in range(5):
            za_v[i, pl.ds(16 * j, 16)] = jnp.zeros((16,), jnp.float32)
        return 0
    lax.fori_loop(0, zrows, _zrow, 0)
    row0 = s * rows_per_tile

    def _zcopy(i, _):
        pltpu.sync_copy(za_v, acc_s.at[pl.ds(row0 + i * zrows, zrows)])
        return 0
    lax.fori_loop(0, rows_per_tile // zrows, _zcopy, 0)
    pltpu.sync_copy(gv_hbm, gv_v)
    plsc.subcore_barrier()

    zero16 = jnp.zeros((16,), jnp.int32)
    iota16 = lax.iota(jnp.int32, 16)
    lane8 = iota16 // 8
    lo8 = iota16 & 7
    g2 = plsc.load_gather(gv_v, [zero16, lo8 + 8])

    def _issue_idx(i, p):
        base = e0 + i * _B
        pltpu.async_copy(src_hbm.at[pl.ds(base, _B)], sidx[p], isem[p])
        pltpu.async_copy(dst_hbm.at[pl.ds(base, _B)], didx[p], isem[p])

    def _wait_idx(p):
        pltpu.make_async_copy(src_hbm.at[pl.ds(0, _B)], sidx[p], isem[p]).wait()
        pltpu.make_async_copy(dst_hbm.at[pl.ds(0, _B)], didx[p], isem[p]).wait()

    def _issue_gather(p):
        pltpu.async_copy(xs_hbm.at[sidx[p]], xr[p], gsem[p])
        pltpu.async_copy(dtab_hbm.at[didx[p]], dt[p], gsem[p])

    def _wait_gather(p):
        pltpu.make_async_copy(xs_hbm.at[sidx[p]], xr[p], gsem[p]).wait()
        pltpu.make_async_copy(dtab_hbm.at[didx[p]], dt[p], gsem[p]).wait()

    def _issue_scatter(p):
        pltpu.async_copy(mw[p], acc_s.at[sdix[p]], ssem[p], add=True)

    def _wait_scatter(p):
        pltpu.make_async_copy(mw[p], acc_s.at[sdix[p]], ssem[p]).wait()

    def _snap_didx(p):
        for k in range(_B // 16):
            sdix[p][pl.ds(16 * k, 16)] = didx[p][pl.ds(16 * k, 16)]

    def _compute(p):
        xr_v, dt_v, mw_v = xr[p], dt[p], mw[p]

        def _edge(e4, _):
            for u in range(2):
                e = e4 * 4 + 2 * u
                epair = e + lane8
                sv = plsc.load_gather(xr_v, [epair, lo8 + _HC])
                dv = plsc.load_gather(dt_v, [epair, lo8])
                t = sv + dv
                t = jnp.maximum(t, 0.2 * t)
                q = dv + g2
                ub = jnp.maximum(q, 0.2 * q)
                w2 = jnp.exp(t - ub)
                plsc.store_scatter(mw_v, [epair, lo8 + _HC], w2)
            for v in range(4):
                e = e4 * 4 + v
                erow = jnp.full((16,), e, jnp.int32)
                for k in range(4):
                    wk = plsc.load_gather(mw_v, [erow, _HC + lane8 + 2 * k])
                    mw_v[e, pl.ds(16 * k, 16)] = xr_v[e, pl.ds(16 * k, 16)] * wk
            return 0
        lax.fori_loop(0, _B // 4, _edge, 0)

    _issue_idx(0, 0)
    _issue_idx(1, 1)
    _wait_idx(0)
    _issue_gather(0)

    def _step(i, p):
        q = 1 - p
        _wait_gather(p)
        _wait_idx(q)
        _issue_gather(q)

        @pl.when(i >= 2)
        def _():
            _wait_scatter(p)
        _snap_didx(p)

        @pl.when(i + 2 < nch)
        def _():
            _issue_idx(i + 2, p)
        _compute(p)
        _issue_scatter(p)

    def _pair(j, _):
        _step(2 * j, 0)
        _step(2 * j + 1, 1)
        return 0
    lax.fori_loop(0, (nch - 1) // 2, _pair, 0)

    _wait_gather(0)
    _wait_scatter(0)
    _snap_didx(0)
    _compute(0)
    _issue_scatter(0)
    _wait_scatter(0)
    _wait_scatter(1)
    plsc.subcore_barrier()

    pltpu.sync_copy(acc_s.at[pl.ds(row0, rows_per_tile)],
                    acc_hbm.at[c, pl.ds(row0, rows_per_tile)])


def _edge_phase(xs, dtab, gv, src, dst):
    rows_per_tile = _NPAD // _NS
    mesh = plsc.VectorSubcoreMesh(core_axis_name="c", subcore_axis_name="s")
    kern = pl.kernel(
        _edge_body,
        out_type=jax.ShapeDtypeStruct((_NC, _NPAD, _ROW), jnp.float32),
        mesh=mesh,
        compiler_params=pltpu.CompilerParams(use_tc_tiling_on_sc=False, needs_layout_passes=False),
        scratch_types=(
            [pltpu.VMEM((_B,), jnp.int32)] * 6
            + [pltpu.VMEM((_B, _ROW), jnp.float32)] * 2
            + [pltpu.VMEM((_B, 16), jnp.float32)] * 2
            + [pltpu.VMEM((_B, _ROW), jnp.float32)] * 2
            + [
                pltpu.VMEM((1, 16), jnp.float32),
                pltpu.VMEM((64, _ROW), jnp.float32),
                pltpu.VMEM_SHARED((_NPAD, _ROW), jnp.float32),
            ]
            + [pltpu.SemaphoreType.DMA] * 6
        ),
    )
    return kern(xs, dtab, gv, src, dst)


def _epilogue_body(acc_ref, bias_ref, out_ref):
    acc = acc_ref[0] + acc_ref[1]
    n = acc.shape[0]
    agg = acc[:, :_HC]
    den = acc[:, _HC:_HC + _HEADS]
    d8 = den.reshape(n, _HEADS, 1) + 1e-16
    a3 = agg.reshape(n, _HEADS, _OUT_C)
    out = (a3 / d8).reshape(n, _HC) + bias_ref[...][None]
    out_ref[...] = jnp.maximum(out, 0.0)


def _epilogue(acc2, bias):
    bn = 1280
    nblk = _NPAD // bn
    return pl.pallas_call(
        _epilogue_body,
        grid=(nblk,),
        in_specs=[
            pl.BlockSpec((_NC, bn, _ROW), lambda i: (0, i, 0)),
            pl.BlockSpec(bias.shape, lambda i: (0,)),
        ],
        out_specs=pl.BlockSpec((bn, _HC), lambda i: (i, 0)),
        out_shape=jax.ShapeDtypeStruct((_NPAD, _HC), jnp.float32),
    )(acc2, bias)


@jax.jit
def kernel(x, edge_index, W, att_src, att_dst, bias):
    src = edge_index[0].astype(jnp.int32)
    dst = edge_index[1].astype(jnp.int32)
    xs, dtab, gv = _prologue(x, W, att_src, att_dst)
    acc2 = _edge_phase(xs, dtab, gv, src, dst)
    out = _epilogue(acc2, bias)
    return out[:x.shape[0]]
